# baseline (speedup 1.0000x reference)
import functools

import jax
import jax.numpy as jnp
from jax import lax
from jax.experimental import pallas as pl
from jax.experimental.pallas import tpu as pltpu
from jax.experimental.pallas import tpu_sc as plsc

_L = 16
_NB = 128
_NG = _NB // _L
_HWORDS = _NB * _L
_N = 8192
_NV = _N // _L
_UNROLL = 8
_RANK = 4096
_NROWS = 2048
_NW = 32
_RPW = _NROWS // _NW
_NBF = float(_NB)
_NB2F = float(_NB * _NB)


def _sc_run(x_hbm, out_hbm, row0_v, row1_v, hist_v, gsum_v, stage_v,
            out_v, sem0, sem1):
    wid = lax.axis_index("s") * 2 + lax.axis_index("c")
    base_row = wid * _RPW
    lane = lax.iota(jnp.int32, _L)
    ones = jnp.ones((_L,), jnp.int32)
    zeros = jnp.zeros((_L,), jnp.int32)

    def zero_hist(i, c):
        hist_v[pl.ds(i * _L, _L)] = zeros
        return c

    lax.fori_loop(0, _NB, zero_hist, 0, unroll=8)

    def fold_scan(rank):
        @plsc.parallel_loop(0, _NG)
        def st1(g):
            base = g * (_L * _L)
            gacc = hist_v[pl.ds(base, _L)]
            for j in range(1, _L):
                gacc = gacc + hist_v[pl.ds(base + j * _L, _L)]
            gsum_v[pl.ds(g * _L, _L)] = gacc

        gtot = [jnp.sum(gsum_v[pl.ds(g * _L, _L)]) for g in range(_NG)]
        cum = jnp.int32(0)
        gstar = jnp.int32(0)
        cumbef = jnp.int32(0)
        for g in range(_NG):
            cum = cum + gtot[g]
            lt = cum < rank
            gstar = gstar + lt.astype(jnp.int32)
            cumbef = jnp.where(lt, cum, cumbef)
        gbase = gstar * (_L * _L)
        cnts = []
        for j in range(_L):
            v = hist_v[pl.ds(gbase + j * _L, _L)]
            cnts.append(jnp.sum(v))

        @plsc.parallel_loop(0, _NB)
        def zz(i):
            hist_v[pl.ds(i * _L, _L)] = zeros

        cum2 = cumbef
        boff = jnp.int32(0)
        cumbef2 = cumbef
        for j in range(_L):
            cum2 = cum2 + cnts[j]
            lt = cum2 < rank
            boff = boff + lt.astype(jnp.int32)
            cumbef2 = jnp.where(lt, cum2, cumbef2)
        bucket = gstar * _L + boff
        return bucket, rank - cumbef2

    def process(row_v, r):
        @plsc.parallel_loop(
            0, _NV, step=_UNROLL,
            carry=tuple(jnp.zeros((_L,), jnp.float32)
                        for _ in range(_UNROLL)))
        def sum_accs(i, accs):
            base = i * _L
            return tuple(accs[k] + row_v[pl.ds(base + k * _L, _L)]
                         for k in range(_UNROLL))

        acc = sum_accs[0]
        for k in range(1, _UNROLL):
            acc = acc + sum_accs[k]

        @plsc.parallel_loop(0, _NV, unroll=16)
        def p1(i):
            v = row_v[pl.ds(i * _L, _L)]
            b = (v * _NBF).astype(jnp.int32)
            plsc.addupdate_scatter(hist_v, [b * _L + lane], ones)

        b1, kk = fold_scan(jnp.int32(_RANK))

        mv = jnp.broadcast_to(b1 * _NB, (_L,)).astype(jnp.float32)

        @plsc.parallel_loop(0, _NV, unroll=16)
        def p2(i):
            v = row_v[pl.ds(i * _L, _L)]
            z = v * _NB2F - mv
            valid = (z >= 0.0) & (z < _NBF)
            b2 = (z).astype(jnp.int32)
            plsc.addupdate_scatter(hist_v, [b2 * _L + lane], ones,
                                   mask=valid)

        b2, _ = fold_scan(kk)

        meanv = jnp.broadcast_to(jnp.sum(acc), (_L,)) * jnp.float32(1.0 / _N)
        q = jnp.broadcast_to(b1 * _NB + b2, (_L,)).astype(jnp.float32)
        mb = q * jnp.float32(1.0 / _NB2F) + jnp.float32(0.5 / _NB2F)
        margin = jnp.abs(mb - 0.5)
        delta = meanv * margin
        outv = jnp.where(mb > 0.5, 0.5 + delta, mb + delta)
        stage_v[pl.ds(r * _L, _L)] = outv

    pltpu.async_copy(x_hbm.at[base_row], row0_v, sem0)

    def pair_step(i, c):
        r0 = i * 2
        pltpu.async_copy(x_hbm.at[base_row + r0 + 1], row1_v, sem1)
        pltpu.make_async_copy(x_hbm.at[base_row + r0], row0_v, sem0).wait()
        process(row0_v, r0)

        @pl.when(r0 + 2 < _RPW)
        def _():
            pltpu.async_copy(x_hbm.at[base_row + r0 + 2], row0_v, sem0)

        pltpu.make_async_copy(x_hbm.at[base_row + r0 + 1], row1_v,
                              sem1).wait()
        process(row1_v, r0 + 1)
        return c

    lax.fori_loop(0, _RPW // 2, pair_step, 0)

    def compact(g, c):
        addr = g * (_L * _L) + lane * _L
        out_v[pl.ds(g * _L, _L)] = plsc.load_gather(stage_v, [addr])
        return c

    lax.fori_loop(0, _RPW // _L, compact, 0)
    pltpu.sync_copy(out_v, out_hbm.at[pl.ds(base_row, _RPW)])


@jax.jit
def kernel(x):
    b, s, n = x.shape
    x2 = x.reshape(b * s, n)
    mesh = plsc.VectorSubcoreMesh(core_axis_name="c", subcore_axis_name="s")
    run = functools.partial(
        pl.kernel,
        mesh=mesh,
        compiler_params=pltpu.CompilerParams(needs_layout_passes=False),
        out_type=jax.ShapeDtypeStruct((_NROWS,), jnp.float32),
        scratch_types=[
            pltpu.VMEM((_N,), jnp.float32),
            pltpu.VMEM((_N,), jnp.float32),
            pltpu.VMEM((_HWORDS,), jnp.int32),
            pltpu.VMEM((_NG * _L,), jnp.int32),
            pltpu.VMEM((_RPW * _L,), jnp.float32),
            pltpu.VMEM((_RPW,), jnp.float32),
            pltpu.SemaphoreType.DMA,
            pltpu.SemaphoreType.DMA,
        ],
    )(_sc_run)
    out = run(x2)
    return out.reshape(b, s)


# speedup vs baseline: 1.1989x; 1.1989x over previous
import functools

import jax
import jax.numpy as jnp
from jax import lax
from jax.experimental import pallas as pl
from jax.experimental.pallas import tpu as pltpu
from jax.experimental.pallas import tpu_sc as plsc

_L = 16
_NB = 128
_NG = _NB // _L
_HWORDS = _NB * _L
_N = 8192
_NV = _N // _L
_UNROLL = 8
_RANK = 4096
_NROWS = 2048
_NW = 32
_RPW = _NROWS // _NW
_NBF = float(_NB)
_NB2F = float(_NB * _NB)


def _sc_run(x_hbm, out_hbm, row0_v, row1_v, hist_v, gsum_v, stage_v,
            out_v, sem0, sem1):
    wid = lax.axis_index("s") * 2 + lax.axis_index("c")
    base_row = wid * _RPW
    lane = lax.iota(jnp.int32, _L)
    ones = jnp.ones((_L,), jnp.int32)
    zeros = jnp.zeros((_L,), jnp.int32)

    def zero_hist(i, c):
        hist_v[pl.ds(i * _L, _L)] = zeros
        return c

    lax.fori_loop(0, _NB, zero_hist, 0, unroll=8)

    def fold_scan(rank):
        @plsc.parallel_loop(0, _NG)
        def st1(g):
            base = g * (_L * _L)
            gacc = hist_v[pl.ds(base, _L)]
            for j in range(1, _L):
                gacc = gacc + hist_v[pl.ds(base + j * _L, _L)]
            gsum_v[pl.ds(g * _L, _L)] = gacc

        gtot = [jnp.sum(gsum_v[pl.ds(g * _L, _L)]) for g in range(_NG)]
        cum = jnp.int32(0)
        gstar = jnp.int32(0)
        cumbef = jnp.int32(0)
        for g in range(_NG):
            cum = cum + gtot[g]
            lt = cum < rank
            gstar = gstar + lt.astype(jnp.int32)
            cumbef = jnp.where(lt, cum, cumbef)
        gbase = gstar * (_L * _L)
        cnts = []
        for j in range(_L):
            v = hist_v[pl.ds(gbase + j * _L, _L)]
            cnts.append(jnp.sum(v))

        @plsc.parallel_loop(0, _NB)
        def zz(i):
            hist_v[pl.ds(i * _L, _L)] = zeros

        cum2 = cumbef
        boff = jnp.int32(0)
        cumbef2 = cumbef
        for j in range(_L):
            cum2 = cum2 + cnts[j]
            lt = cum2 < rank
            boff = boff + lt.astype(jnp.int32)
            cumbef2 = jnp.where(lt, cum2, cumbef2)
        bucket = gstar * _L + boff
        return bucket, rank - cumbef2

    def process(row_v, r):
        @plsc.parallel_loop(
            0, _NV, step=_UNROLL,
            carry=tuple(jnp.zeros((_L,), jnp.float32)
                        for _ in range(_UNROLL)))
        def sum_accs(i, accs):
            base = i * _L
            return tuple(accs[k] + row_v[pl.ds(base + k * _L, _L)]
                         for k in range(_UNROLL))

        acc = sum_accs[0]
        for k in range(1, _UNROLL):
            acc = acc + sum_accs[k]

        @plsc.parallel_loop(0, _NV, unroll=_UNROLL)
        def p1(i):
            v = row_v[pl.ds(i * _L, _L)]
            b = (v * _NBF).astype(jnp.int32)
            plsc.addupdate_scatter(hist_v, [b * _L + lane], ones)

        b1, kk = fold_scan(jnp.int32(_RANK))

        mv = jnp.broadcast_to(b1 * _NB, (_L,)).astype(jnp.float32)

        @plsc.parallel_loop(0, _NV, unroll=_UNROLL)
        def p2(i):
            v = row_v[pl.ds(i * _L, _L)]
            z = v * _NB2F - mv
            valid = (z >= 0.0) & (z < _NBF)
            b2 = (z).astype(jnp.int32)
            plsc.addupdate_scatter(hist_v, [b2 * _L + lane], ones,
                                   mask=valid)

        b2, _ = fold_scan(kk)

        meanv = jnp.broadcast_to(jnp.sum(acc), (_L,)) * jnp.float32(1.0 / _N)
        q = jnp.broadcast_to(b1 * _NB + b2, (_L,)).astype(jnp.float32)
        mb = q * jnp.float32(1.0 / _NB2F) + jnp.float32(0.5 / _NB2F)
        margin = jnp.abs(mb - 0.5)
        delta = meanv * margin
        outv = jnp.where(mb > 0.5, 0.5 + delta, mb + delta)
        stage_v[pl.ds(r * _L, _L)] = outv

    pltpu.async_copy(x_hbm.at[base_row], row0_v, sem0)

    def pair_step(i, c):
        r0 = i * 2
        pltpu.async_copy(x_hbm.at[base_row + r0 + 1], row1_v, sem1)
        pltpu.make_async_copy(x_hbm.at[base_row + r0], row0_v, sem0).wait()
        process(row0_v, r0)

        @pl.when(r0 + 2 < _RPW)
        def _():
            pltpu.async_copy(x_hbm.at[base_row + r0 + 2], row0_v, sem0)

        pltpu.make_async_copy(x_hbm.at[base_row + r0 + 1], row1_v,
                              sem1).wait()
        process(row1_v, r0 + 1)
        return c

    lax.fori_loop(0, _RPW // 2, pair_step, 0)

    def compact(g, c):
        addr = g * (_L * _L) + lane * _L
        out_v[pl.ds(g * _L, _L)] = plsc.load_gather(stage_v, [addr])
        return c

    lax.fori_loop(0, _RPW // _L, compact, 0)
    pltpu.sync_copy(out_v, out_hbm.at[pl.ds(base_row, _RPW)])


@jax.jit
def kernel(x):
    b, s, n = x.shape
    x2 = x.reshape(b * s, n)
    mesh = plsc.VectorSubcoreMesh(core_axis_name="c", subcore_axis_name="s")
    run = functools.partial(
        pl.kernel,
        mesh=mesh,
        compiler_params=pltpu.CompilerParams(needs_layout_passes=False),
        out_type=jax.ShapeDtypeStruct((_NROWS,), jnp.float32),
        scratch_types=[
            pltpu.VMEM((_N,), jnp.float32),
            pltpu.VMEM((_N,), jnp.float32),
            pltpu.VMEM((_HWORDS,), jnp.int32),
            pltpu.VMEM((_NG * _L,), jnp.int32),
            pltpu.VMEM((_RPW * _L,), jnp.float32),
            pltpu.VMEM((_RPW,), jnp.float32),
            pltpu.SemaphoreType.DMA,
            pltpu.SemaphoreType.DMA,
        ],
    )(_sc_run)
    out = run(x2)
    return out.reshape(b, s)


# speedup vs baseline: 1.7738x; 1.4795x over previous
import functools

import jax
import jax.numpy as jnp
from jax import lax
from jax.experimental import pallas as pl
from jax.experimental.pallas import tpu as pltpu
from jax.experimental.pallas import tpu_sc as plsc

_L = 16
_NB = 128
_NG = _NB // _L
_HWORDS = _NB * _L
_N = 8192
_NV = _N // _L
_UNROLL = 8
_RANK = 4096
_NROWS = 2048
_NW = 32
_RPW = _NROWS // _NW
_NBF = float(_NB)
_NB2F = float(_NB * _NB)


def _sc_run(x_hbm, out_hbm, row0_v, row1_v, hist_v, stage_v, out_v,
            sem0, sem1):
    wid = lax.axis_index("s") * 2 + lax.axis_index("c")
    base_row = wid * _RPW
    lane = lax.iota(jnp.int32, _L)
    ones = jnp.ones((_L,), jnp.int32)
    zeros = jnp.zeros((_L,), jnp.int32)

    def zero_hist(i, c):
        hist_v[pl.ds(i * _L, _L)] = zeros
        return c

    lax.fori_loop(0, _NB, zero_hist, 0, unroll=8)

    def fold_scan(rank):
        gtot = []
        for g in range(_NG):
            gacc = hist_v[pl.ds(g * 256, _L)]
            for j in range(1, _L):
                gacc = gacc + hist_v[pl.ds(g * 256 + j * _L, _L)]
            gtot.append(jnp.sum(gacc))
        cum = jnp.int32(0)
        gstar = jnp.int32(0)
        cumbef = jnp.int32(0)
        for g in range(_NG):
            cum = cum + gtot[g]
            lt = cum < rank
            gstar = gstar + lt.astype(jnp.int32)
            cumbef = jnp.where(lt, cum, cumbef)
        gbase = gstar * 256
        cnts = []
        for j in range(_L):
            v = hist_v[pl.ds(gbase + j * _L, _L)]
            cnts.append(jnp.sum(v))
        for g in range(_NG):
            for j in range(_L):
                hist_v[pl.ds(g * 256 + j * _L, _L)] = zeros
        cum2 = cumbef
        boff = jnp.int32(0)
        cumbef2 = cumbef
        for j in range(_L):
            cum2 = cum2 + cnts[j]
            lt = cum2 < rank
            boff = boff + lt.astype(jnp.int32)
            cumbef2 = jnp.where(lt, cum2, cumbef2)
        bucket = gstar * _L + boff
        return bucket, rank - cumbef2

    def process(row_v, r):
        @plsc.parallel_loop(
            0, _NV, step=2, unroll=4,
            carry=(jnp.zeros((_L,), jnp.float32),
                   jnp.zeros((_L,), jnp.float32)))
        def p1_accs(i, accs):
            a0, a1 = accs
            v0 = row_v[pl.ds(i * _L, _L)]
            v1 = row_v[pl.ds((i + 1) * _L, _L)]
            b0 = (v0 * _NBF).astype(jnp.int32)
            b1i = (v1 * _NBF).astype(jnp.int32)
            plsc.addupdate_scatter(hist_v, [b0 * _L + lane], ones)
            plsc.addupdate_scatter(hist_v, [b1i * _L + lane], ones)
            return (a0 + v0, a1 + v1)

        acc = p1_accs[0] + p1_accs[1]

        b1, kk = fold_scan(jnp.int32(_RANK))

        mv = jnp.broadcast_to(b1 * _NB, (_L,)).astype(jnp.float32)

        @plsc.parallel_loop(0, _NV, unroll=_UNROLL)
        def p2(i):
            v = row_v[pl.ds(i * _L, _L)]
            z = v * _NB2F - mv
            valid = (z >= 0.0) & (z < _NBF)
            b2 = (z).astype(jnp.int32)
            plsc.addupdate_scatter(hist_v, [b2 * _L + lane], ones,
                                   mask=valid)

        b2, _ = fold_scan(kk)

        meanv = jnp.broadcast_to(jnp.sum(acc), (_L,)) * jnp.float32(1.0 / _N)
        q = jnp.broadcast_to(b1 * _NB + b2, (_L,)).astype(jnp.float32)
        mb = q * jnp.float32(1.0 / _NB2F) + jnp.float32(0.5 / _NB2F)
        margin = jnp.abs(mb - 0.5)
        delta = meanv * margin
        outv = jnp.where(mb > 0.5, 0.5 + delta, mb + delta)
        stage_v[pl.ds(r * _L, _L)] = outv

    pltpu.async_copy(x_hbm.at[base_row], row0_v, sem0)

    def pair_step(i, c):
        r0 = i * 2
        pltpu.async_copy(x_hbm.at[base_row + r0 + 1], row1_v, sem1)
        pltpu.make_async_copy(x_hbm.at[base_row + r0], row0_v, sem0).wait()
        process(row0_v, r0)

        @pl.when(r0 + 2 < _RPW)
        def _():
            pltpu.async_copy(x_hbm.at[base_row + r0 + 2], row0_v, sem0)

        pltpu.make_async_copy(x_hbm.at[base_row + r0 + 1], row1_v,
                              sem1).wait()
        process(row1_v, r0 + 1)
        return c

    lax.fori_loop(0, _RPW // 2, pair_step, 0)

    def compact(g, c):
        addr = g * (_L * _L) + lane * _L
        out_v[pl.ds(g * _L, _L)] = plsc.load_gather(stage_v, [addr])
        return c

    lax.fori_loop(0, _RPW // _L, compact, 0)
    pltpu.sync_copy(out_v, out_hbm.at[pl.ds(base_row, _RPW)])


@jax.jit
def kernel(x):
    b, s, n = x.shape
    x2 = x.reshape(b * s, n)
    mesh = plsc.VectorSubcoreMesh(core_axis_name="c", subcore_axis_name="s")
    run = functools.partial(
        pl.kernel,
        mesh=mesh,
        compiler_params=pltpu.CompilerParams(needs_layout_passes=False),
        out_type=jax.ShapeDtypeStruct((_NROWS,), jnp.float32),
        scratch_types=[
            pltpu.VMEM((_N,), jnp.float32),
            pltpu.VMEM((_N,), jnp.float32),
            pltpu.VMEM((_HWORDS,), jnp.int32),
            pltpu.VMEM((_RPW * _L,), jnp.float32),
            pltpu.VMEM((_RPW,), jnp.float32),
            pltpu.SemaphoreType.DMA,
            pltpu.SemaphoreType.DMA,
        ],
    )(_sc_run)
    out = run(x2)
    return out.reshape(b, s)


# speedup vs baseline: 2.1205x; 1.1954x over previous
import functools

import jax
import jax.numpy as jnp
from jax import lax
from jax.experimental import pallas as pl
from jax.experimental.pallas import tpu as pltpu
from jax.experimental.pallas import tpu_sc as plsc

_L = 16
_NB = 64
_NG = _NB // _L
_HWORDS = _NB * _L
_N = 8192
_NV = _N // _L
_UNROLL = 8
_RANK = 4096
_NROWS = 2048
_NW = 32
_RPW = _NROWS // _NW
_NBF = float(_NB)
_NB2F = float(_NB * _NB)


def _sc_run(x_hbm, out_hbm, row0_v, row1_v, hist_v, stage_v, out_v,
            sem0, sem1):
    wid = lax.axis_index("s") * 2 + lax.axis_index("c")
    base_row = wid * _RPW
    lane = lax.iota(jnp.int32, _L)
    ones = jnp.ones((_L,), jnp.int32)
    zeros = jnp.zeros((_L,), jnp.int32)

    def zero_hist(i, c):
        hist_v[pl.ds(i * _L, _L)] = zeros
        return c

    lax.fori_loop(0, _NB, zero_hist, 0, unroll=8)

    def fold_scan(rank):
        gtot = []
        for g in range(_NG):
            gacc = hist_v[pl.ds(g * 256, _L)]
            for j in range(1, _L):
                gacc = gacc + hist_v[pl.ds(g * 256 + j * _L, _L)]
            gtot.append(jnp.sum(gacc))
        cum = jnp.int32(0)
        gstar = jnp.int32(0)
        cumbef = jnp.int32(0)
        for g in range(_NG):
            cum = cum + gtot[g]
            lt = cum < rank
            gstar = gstar + lt.astype(jnp.int32)
            cumbef = jnp.where(lt, cum, cumbef)
        gbase = gstar * 256
        cnts = []
        for j in range(_L):
            v = hist_v[pl.ds(gbase + j * _L, _L)]
            cnts.append(jnp.sum(v))
        for g in range(_NG):
            for j in range(_L):
                hist_v[pl.ds(g * 256 + j * _L, _L)] = zeros
        cum2 = cumbef
        boff = jnp.int32(0)
        cumbef2 = cumbef
        for j in range(_L):
            cum2 = cum2 + cnts[j]
            lt = cum2 < rank
            boff = boff + lt.astype(jnp.int32)
            cumbef2 = jnp.where(lt, cum2, cumbef2)
        bucket = gstar * _L + boff
        return bucket, rank - cumbef2

    def process(row_v, r):
        @plsc.parallel_loop(
            0, _NV, step=2, unroll=4,
            carry=(jnp.zeros((_L,), jnp.float32),
                   jnp.zeros((_L,), jnp.float32)))
        def p1_accs(i, accs):
            a0, a1 = accs
            v0 = row_v[pl.ds(i * _L, _L)]
            v1 = row_v[pl.ds((i + 1) * _L, _L)]
            b0 = (v0 * _NBF).astype(jnp.int32)
            b1i = (v1 * _NBF).astype(jnp.int32)
            plsc.addupdate_scatter(hist_v, [b0 * _L + lane], ones)
            plsc.addupdate_scatter(hist_v, [b1i * _L + lane], ones)
            return (a0 + v0, a1 + v1)

        acc = p1_accs[0] + p1_accs[1]

        b1, kk = fold_scan(jnp.int32(_RANK))

        mv = jnp.broadcast_to(b1 * _NB, (_L,)).astype(jnp.float32)

        @plsc.parallel_loop(0, _NV, unroll=_UNROLL)
        def p2(i):
            v = row_v[pl.ds(i * _L, _L)]
            z = v * _NB2F - mv
            valid = (z >= 0.0) & (z < _NBF)
            b2 = (z).astype(jnp.int32)
            plsc.addupdate_scatter(hist_v, [b2 * _L + lane], ones,
                                   mask=valid)

        b2, _ = fold_scan(kk)

        meanv = jnp.broadcast_to(jnp.sum(acc), (_L,)) * jnp.float32(1.0 / _N)
        q = jnp.broadcast_to(b1 * _NB + b2, (_L,)).astype(jnp.float32)
        mb = q * jnp.float32(1.0 / _NB2F) + jnp.float32(0.5 / _NB2F)
        margin = jnp.abs(mb - 0.5)
        delta = meanv * margin
        outv = jnp.where(mb > 0.5, 0.5 + delta, mb + delta)
        stage_v[pl.ds(r * _L, _L)] = outv

    pltpu.async_copy(x_hbm.at[base_row], row0_v, sem0)

    def pair_step(i, c):
        r0 = i * 2
        pltpu.async_copy(x_hbm.at[base_row + r0 + 1], row1_v, sem1)
        pltpu.make_async_copy(x_hbm.at[base_row + r0], row0_v, sem0).wait()
        process(row0_v, r0)

        @pl.when(r0 + 2 < _RPW)
        def _():
            pltpu.async_copy(x_hbm.at[base_row + r0 + 2], row0_v, sem0)

        pltpu.make_async_copy(x_hbm.at[base_row + r0 + 1], row1_v,
                              sem1).wait()
        process(row1_v, r0 + 1)
        return c

    lax.fori_loop(0, _RPW // 2, pair_step, 0)

    def compact(g, c):
        addr = g * (_L * _L) + lane * _L
        out_v[pl.ds(g * _L, _L)] = plsc.load_gather(stage_v, [addr])
        return c

    lax.fori_loop(0, _RPW // _L, compact, 0)
    pltpu.sync_copy(out_v, out_hbm.at[pl.ds(base_row, _RPW)])


@jax.jit
def kernel(x):
    b, s, n = x.shape
    x2 = x.reshape(b * s, n)
    mesh = plsc.VectorSubcoreMesh(core_axis_name="c", subcore_axis_name="s")
    run = functools.partial(
        pl.kernel,
        mesh=mesh,
        compiler_params=pltpu.CompilerParams(needs_layout_passes=False),
        out_type=jax.ShapeDtypeStruct((_NROWS,), jnp.float32),
        scratch_types=[
            pltpu.VMEM((_N,), jnp.float32),
            pltpu.VMEM((_N,), jnp.float32),
            pltpu.VMEM((_HWORDS,), jnp.int32),
            pltpu.VMEM((_RPW * _L,), jnp.float32),
            pltpu.VMEM((_RPW,), jnp.float32),
            pltpu.SemaphoreType.DMA,
            pltpu.SemaphoreType.DMA,
        ],
    )(_sc_run)
    out = run(x2)
    return out.reshape(b, s)


# speedup vs baseline: 2.1558x; 1.0166x over previous
import functools

import jax
import jax.numpy as jnp
from jax import lax
from jax.experimental import pallas as pl
from jax.experimental.pallas import tpu as pltpu
from jax.experimental.pallas import tpu_sc as plsc

_L = 16
_NB = 64
_NG = _NB // _L
_HWORDS = _NB * _L
_N = 8192
_NV = _N // _L
_UNROLL = 8
_RANK = 4096
_NROWS = 2048
_NW = 32
_RPW = _NROWS // _NW
_NBF = float(_NB)
_NB2F = float(_NB * _NB)


def _sc_run(x_hbm, out_hbm, row0_v, row1_v, hist_v, stage_v, out_v,
            sem0, sem1):
    wid = lax.axis_index("s") * 2 + lax.axis_index("c")
    base_row = wid * _RPW
    lane = lax.iota(jnp.int32, _L)
    lane2 = lane - _NB * _L
    ones = jnp.ones((_L,), jnp.int32)
    zeros = jnp.zeros((_L,), jnp.int32)

    def zero_hist(i, c):
        hist_v[pl.ds(i * _L, _L)] = zeros
        return c

    lax.fori_loop(0, _NB, zero_hist, 0, unroll=8)

    def fold_scan(rank):
        gtot = []
        for g in range(_NG):
            gacc = hist_v[pl.ds(g * 256, _L)]
            for j in range(1, _L):
                gacc = gacc + hist_v[pl.ds(g * 256 + j * _L, _L)]
            gtot.append(jnp.sum(gacc))
        cum = jnp.int32(0)
        gstar = jnp.int32(0)
        cumbef = jnp.int32(0)
        for g in range(_NG):
            cum = cum + gtot[g]
            lt = cum < rank
            gstar = gstar + lt.astype(jnp.int32)
            cumbef = jnp.where(lt, cum, cumbef)
        gbase = gstar * 256
        cnts = []
        for j in range(_L):
            v = hist_v[pl.ds(gbase + j * _L, _L)]
            cnts.append(jnp.sum(v))
        for g in range(_NG):
            for j in range(_L):
                hist_v[pl.ds(g * 256 + j * _L, _L)] = zeros
        cum2 = cumbef
        boff = jnp.int32(0)
        cumbef2 = cumbef
        for j in range(_L):
            cum2 = cum2 + cnts[j]
            lt = cum2 < rank
            boff = boff + lt.astype(jnp.int32)
            cumbef2 = jnp.where(lt, cum2, cumbef2)
        bucket = gstar * _L + boff
        return bucket, rank - cumbef2

    def process(row_v, r):
        @plsc.parallel_loop(
            0, _NV, step=2, unroll=4,
            carry=(jnp.zeros((_L,), jnp.float32),
                   jnp.zeros((_L,), jnp.float32)))
        def p1_accs(i, accs):
            a0, a1 = accs
            v0 = row_v[pl.ds(i * _L, _L)]
            v1 = row_v[pl.ds((i + 1) * _L, _L)]
            b0 = (v0 * _NBF).astype(jnp.int32)
            b1i = (v1 * _NBF).astype(jnp.int32)
            plsc.addupdate_scatter(hist_v, [b0 * _L + lane], ones)
            plsc.addupdate_scatter(hist_v, [b1i * _L + lane], ones)
            return (a0 + v0, a1 + v1)

        acc = p1_accs[0] + p1_accs[1]

        b1, kk = fold_scan(jnp.int32(_RANK))

        mv = jnp.broadcast_to((b1 - 1) * _NB, (_L,)).astype(jnp.float32)

        @plsc.parallel_loop(0, _NV, unroll=_UNROLL)
        def p2(i):
            v = row_v[pl.ds(i * _L, _L)]
            z = v * _NB2F - mv
            b2s = z.astype(jnp.int32)
            valid = (b2s - _NB).astype(jnp.uint32) < jnp.uint32(_NB)
            plsc.addupdate_scatter(hist_v, [b2s * _L + lane2], ones,
                                   mask=valid)

        b2, _ = fold_scan(kk)

        meanv = jnp.broadcast_to(jnp.sum(acc), (_L,)) * jnp.float32(1.0 / _N)
        q = jnp.broadcast_to(b1 * _NB + b2, (_L,)).astype(jnp.float32)
        mb = q * jnp.float32(1.0 / _NB2F) + jnp.float32(0.5 / _NB2F)
        margin = jnp.abs(mb - 0.5)
        delta = meanv * margin
        outv = jnp.where(mb > 0.5, 0.5 + delta, mb + delta)
        stage_v[pl.ds(r * _L, _L)] = outv

    pltpu.async_copy(x_hbm.at[base_row], row0_v, sem0)

    def pair_step(i, c):
        r0 = i * 2
        pltpu.async_copy(x_hbm.at[base_row + r0 + 1], row1_v, sem1)
        pltpu.make_async_copy(x_hbm.at[base_row + r0], row0_v, sem0).wait()
        process(row0_v, r0)

        @pl.when(r0 + 2 < _RPW)
        def _():
            pltpu.async_copy(x_hbm.at[base_row + r0 + 2], row0_v, sem0)

        pltpu.make_async_copy(x_hbm.at[base_row + r0 + 1], row1_v,
                              sem1).wait()
        process(row1_v, r0 + 1)
        return c

    lax.fori_loop(0, _RPW // 2, pair_step, 0)

    def compact(g, c):
        addr = g * (_L * _L) + lane * _L
        out_v[pl.ds(g * _L, _L)] = plsc.load_gather(stage_v, [addr])
        return c

    lax.fori_loop(0, _RPW // _L, compact, 0)
    pltpu.sync_copy(out_v, out_hbm.at[pl.ds(base_row, _RPW)])


@jax.jit
def kernel(x):
    b, s, n = x.shape
    x2 = x.reshape(b * s, n)
    mesh = plsc.VectorSubcoreMesh(core_axis_name="c", subcore_axis_name="s")
    run = functools.partial(
        pl.kernel,
        mesh=mesh,
        compiler_params=pltpu.CompilerParams(needs_layout_passes=False),
        out_type=jax.ShapeDtypeStruct((_NROWS,), jnp.float32),
        scratch_types=[
            pltpu.VMEM((_N,), jnp.float32),
            pltpu.VMEM((_N,), jnp.float32),
            pltpu.VMEM((_HWORDS,), jnp.int32),
            pltpu.VMEM((_RPW * _L,), jnp.float32),
            pltpu.VMEM((_RPW,), jnp.float32),
            pltpu.SemaphoreType.DMA,
            pltpu.SemaphoreType.DMA,
        ],
    )(_sc_run)
    out = run(x2)
    return out.reshape(b, s)
